# baseline (speedup 1.0000x reference)
import jax
import jax.numpy as jnp
from jax.experimental import pallas as pl
from jax.experimental.pallas import tpu as pltpu

N = 100000
D_IN = 56
NUM_GRAPHS = 512
ROWS = 10000
STEPS = N // ROWS
HI = jax.lax.Precision.HIGHEST
DN0 = (((0,), (0,)), ((), ()))
F32 = jnp.float32
BF16 = jnp.bfloat16


def _fused_kernel(x_ref, batch_ref,
                  a1, c1, W1h, b1,
                  W2, b2, bn2_g, bn2_b, bn2_m, bn2_v,
                  W3, b3, bn3_g, bn3_b, bn3_m, bn3_v,
                  W4, b4, bn4_g, bn4_b, bn4_m, bn4_v,
                  out_ref, acc_ref):
    i = pl.program_id(0)

    @pl.when(i == 0)
    def _init():
        acc_ref[...] = jnp.zeros_like(acc_ref)

    t = x_ref[...] * a1[...] + c1[...]
    h = jnp.where(t > 0, t, jnp.exp(t) - 1.0)

    h_hi = h.astype(BF16)
    h_lo = (h - h_hi.astype(F32)).astype(BF16)
    h1 = (jnp.dot(h_hi, W1h[...], preferred_element_type=F32)
          + jnp.dot(h_lo, W1h[...], preferred_element_type=F32)
          + b1[...])

    p0 = h1.astype(BF16)
    r1 = h1 - p0.astype(F32)
    p1 = r1.astype(BF16)
    p2 = (r1 - p1.astype(F32)).astype(BF16)
    seg = batch_ref[0]
    gid = jax.lax.broadcasted_iota(jnp.int16, (ROWS, NUM_GRAPHS), 1)
    onehot = jnp.where(seg == gid, BF16(1), BF16(0))
    acc_ref[...] += (
        jax.lax.dot_general(p0, onehot, DN0, preferred_element_type=F32)
        + jax.lax.dot_general(p1, onehot, DN0, preferred_element_type=F32)
        + jax.lax.dot_general(p2, onehot, DN0, preferred_element_type=F32))

    @pl.when(i == STEPS - 1)
    def _epilogue():
        pooled = acc_ref[...]
        z = jax.lax.dot_general(pooled, W2[...], DN0,
                                precision=HI, preferred_element_type=F32)
        z += b2[...]
        z = (z - bn2_m[...]) / jnp.sqrt(bn2_v[...] + 1e-5)
        z = jnp.maximum(z * bn2_g[...] + bn2_b[...], 0.0)
        z = jnp.dot(z, W3[...], precision=HI, preferred_element_type=F32)
        z += b3[...]
        z = (z - bn3_m[...]) / jnp.sqrt(bn3_v[...] + 1e-5)
        z = jnp.maximum(z * bn3_g[...] + bn3_b[...], 0.0)
        z = jnp.dot(z, W4[...], precision=HI, preferred_element_type=F32)
        z += b4[...]
        z = (z - bn4_m[...]) / jnp.sqrt(bn4_v[...] + 1e-5)
        out_ref[...] = z * bn4_g[...] + bn4_b[...]


def kernel(x, edge_index, batch,
           bn1_g, bn1_b, bn1_m, bn1_v, W1, b1,
           W2, b2, bn2_g, bn2_b, bn2_m, bn2_v,
           W3, b3, bn3_g, bn3_b, bn3_m, bn3_v,
           W4, b4, bn4_g, bn4_b, bn4_m, bn4_v):
    del edge_index
    batch3 = batch.astype(jnp.int16).reshape(STEPS, ROWS, 1)
    W1h = W1.astype(BF16)
    W2 = W2.astype(BF16).astype(jnp.float32)
    W3 = W3.astype(BF16).astype(jnp.float32)
    W4 = W4.astype(BF16).astype(jnp.float32)
    a1 = bn1_g * jax.lax.rsqrt(bn1_v + 1e-5)
    c1 = bn1_b - bn1_m * a1
    row = lambda v: v.reshape(1, -1)

    full = lambda shape: pl.BlockSpec(shape, lambda i: (0,) * len(shape))
    out = pl.pallas_call(
        _fused_kernel,
        grid=(STEPS,),
        in_specs=[
            pl.BlockSpec((ROWS, D_IN), lambda i: (i, 0)),
            pl.BlockSpec((1, ROWS, 1), lambda i: (i, 0, 0)),
            full((1, D_IN)), full((1, D_IN)),
            full(W1.shape), full((1, 64)),
            full(W2.shape), full((1, 128)),
            full((1, 128)), full((1, 128)), full((1, 128)), full((1, 128)),
            full(W3.shape), full((1, 64)),
            full((1, 64)), full((1, 64)), full((1, 64)), full((1, 64)),
            full(W4.shape), full((1, 1)),
            full((1, 1)), full((1, 1)), full((1, 1)), full((1, 1)),
        ],
        out_specs=pl.BlockSpec((NUM_GRAPHS, 1), lambda i: (0, 0)),
        out_shape=jax.ShapeDtypeStruct((NUM_GRAPHS, 1), jnp.float32),
        scratch_shapes=[
            pltpu.VMEM((64, NUM_GRAPHS), jnp.float32),
        ],
    )(x, batch3,
      row(a1), row(c1), W1h, row(b1),
      W2, row(b2), row(bn2_g), row(bn2_b), row(bn2_m), row(bn2_v),
      W3, row(b3), row(bn3_g), row(bn3_b), row(bn3_m), row(bn3_v),
      W4, row(b4), row(bn4_g), row(bn4_b), row(bn4_m), row(bn4_v))
    return out.reshape(-1)


# speedup vs baseline: 1.0033x; 1.0033x over previous
import jax
import jax.numpy as jnp
from jax.experimental import pallas as pl
from jax.experimental.pallas import tpu as pltpu

N = 100000
D_IN = 56
NUM_GRAPHS = 512
ROWS = 5000
STEPS = N // ROWS
HI = jax.lax.Precision.HIGHEST
DN0 = (((0,), (0,)), ((), ()))
F32 = jnp.float32
BF16 = jnp.bfloat16


def _fused_kernel(x_ref, batch_ref,
                  a1, c1, W1h, b1,
                  W2, b2, bn2_g, bn2_b, bn2_m, bn2_v,
                  W3, b3, bn3_g, bn3_b, bn3_m, bn3_v,
                  W4, b4, bn4_g, bn4_b, bn4_m, bn4_v,
                  out_ref, acc_ref):
    i = pl.program_id(0)

    @pl.when(i == 0)
    def _init():
        acc_ref[...] = jnp.zeros_like(acc_ref)

    t = x_ref[...] * a1[...] + c1[...]
    h = jnp.where(t > 0, t, jnp.exp(t) - 1.0)

    h_hi = h.astype(BF16)
    h_lo = (h - h_hi.astype(F32)).astype(BF16)
    h1 = (jnp.dot(h_hi, W1h[...], preferred_element_type=F32)
          + jnp.dot(h_lo, W1h[...], preferred_element_type=F32)
          + b1[...])

    p0 = h1.astype(BF16)
    r1 = h1 - p0.astype(F32)
    p1 = r1.astype(BF16)
    p2 = (r1 - p1.astype(F32)).astype(BF16)
    seg = batch_ref[0]
    gid = jax.lax.broadcasted_iota(jnp.int16, (ROWS, NUM_GRAPHS), 1)
    onehot = jnp.where(seg == gid, BF16(1), BF16(0))
    acc_ref[...] += (
        jax.lax.dot_general(p0, onehot, DN0, preferred_element_type=F32)
        + jax.lax.dot_general(p1, onehot, DN0, preferred_element_type=F32)
        + jax.lax.dot_general(p2, onehot, DN0, preferred_element_type=F32))

    @pl.when(i == STEPS - 1)
    def _epilogue():
        pooled = acc_ref[...]
        z = jax.lax.dot_general(pooled, W2[...], DN0,
                                precision=HI, preferred_element_type=F32)
        z += b2[...]
        z = (z - bn2_m[...]) / jnp.sqrt(bn2_v[...] + 1e-5)
        z = jnp.maximum(z * bn2_g[...] + bn2_b[...], 0.0)
        z = jnp.dot(z, W3[...], precision=HI, preferred_element_type=F32)
        z += b3[...]
        z = (z - bn3_m[...]) / jnp.sqrt(bn3_v[...] + 1e-5)
        z = jnp.maximum(z * bn3_g[...] + bn3_b[...], 0.0)
        z = jnp.dot(z, W4[...], precision=HI, preferred_element_type=F32)
        z += b4[...]
        z = (z - bn4_m[...]) / jnp.sqrt(bn4_v[...] + 1e-5)
        out_ref[...] = z * bn4_g[...] + bn4_b[...]


def kernel(x, edge_index, batch,
           bn1_g, bn1_b, bn1_m, bn1_v, W1, b1,
           W2, b2, bn2_g, bn2_b, bn2_m, bn2_v,
           W3, b3, bn3_g, bn3_b, bn3_m, bn3_v,
           W4, b4, bn4_g, bn4_b, bn4_m, bn4_v):
    del edge_index
    batch3 = batch.astype(jnp.int16).reshape(STEPS, ROWS, 1)
    W1h = W1.astype(BF16)
    W2 = W2.astype(BF16).astype(jnp.float32)
    W3 = W3.astype(BF16).astype(jnp.float32)
    W4 = W4.astype(BF16).astype(jnp.float32)
    a1 = bn1_g * jax.lax.rsqrt(bn1_v + 1e-5)
    c1 = bn1_b - bn1_m * a1
    row = lambda v: v.reshape(1, -1)

    full = lambda shape: pl.BlockSpec(shape, lambda i: (0,) * len(shape))
    out = pl.pallas_call(
        _fused_kernel,
        grid=(STEPS,),
        in_specs=[
            pl.BlockSpec((ROWS, D_IN), lambda i: (i, 0)),
            pl.BlockSpec((1, ROWS, 1), lambda i: (i, 0, 0)),
            full((1, D_IN)), full((1, D_IN)),
            full(W1.shape), full((1, 64)),
            full(W2.shape), full((1, 128)),
            full((1, 128)), full((1, 128)), full((1, 128)), full((1, 128)),
            full(W3.shape), full((1, 64)),
            full((1, 64)), full((1, 64)), full((1, 64)), full((1, 64)),
            full(W4.shape), full((1, 1)),
            full((1, 1)), full((1, 1)), full((1, 1)), full((1, 1)),
        ],
        out_specs=pl.BlockSpec((NUM_GRAPHS, 1), lambda i: (0, 0)),
        out_shape=jax.ShapeDtypeStruct((NUM_GRAPHS, 1), jnp.float32),
        scratch_shapes=[
            pltpu.VMEM((64, NUM_GRAPHS), jnp.float32),
        ],
    )(x, batch3,
      row(a1), row(c1), W1h, row(b1),
      W2, row(b2), row(bn2_g), row(bn2_b), row(bn2_m), row(bn2_v),
      W3, row(b3), row(bn3_g), row(bn3_b), row(bn3_m), row(bn3_v),
      W4, row(b4), row(bn4_g), row(bn4_b), row(bn4_m), row(bn4_v))
    return out.reshape(-1)
